# baseline (speedup 1.0000x reference)
import functools

import jax
import jax.numpy as jnp
from jax import lax
from jax.experimental import pallas as pl
from jax.experimental.pallas import tpu as pltpu
from jax.experimental.pallas import tpu_sc as plsc

N = 10000
E = 320000
D = 128
CK = 128
ER = E // CK
TABN = 10240
NC = 2
NS = 16
ROWS_PER_TILE = TABN // NS
TPT = ER // NS
TEX = ER - NS * TPT
HR = ER // NC
HPT = HR // NS
HEX = HR - NS * HPT
P1 = 26
K2 = 3
DEGW = 6

_mesh = plsc.VectorSubcoreMesh(core_axis_name="c", subcore_axis_name="s")
_sc_params = pltpu.CompilerParams(use_tc_tiling_on_sc=False)

_f32 = jnp.float32


def _fill(ref, rows, cols, value):
    vec = jnp.full((16,), value, dtype=_f32)

    @pl.loop(0, rows)
    def _(r):
        @pl.loop(0, cols, step=16)
        def _(k):
            ref[r, pl.ds(k, 16)] = vec


@functools.partial(
    pl.kernel,
    out_type=jax.ShapeDtypeStruct((NC, TABN, 16), _f32),
    mesh=_mesh,
    compiler_params=_sc_params,
    scratch_types=[
        pltpu.VMEM((TPT, CK), jnp.int32),
        pltpu.VMEM((1, CK), jnp.int32),
        pltpu.VMEM((CK, 16), _f32),
        pltpu.VMEM((ROWS_PER_TILE, 16), _f32),
        pltpu.VMEM_SHARED((TABN, 16), _f32),
        pltpu.SemaphoreType.DMA((DEGW,)),
        pltpu.SemaphoreType.DMA,
    ],
)
def _deg_kernel(eidx_hbm, deg_hbm, idx_v, exi_v, ones_v, zbuf_v, table_sh, ssem, isem):
    c = lax.axis_index("c")
    s = lax.axis_index("s")
    pltpu.async_copy(eidx_hbm.at[c, pl.ds(s * TPT, TPT)], idx_v, isem)
    _fill(ones_v, CK, 16, 1.0)
    _fill(zbuf_v, ROWS_PER_TILE, 16, 0.0)
    pltpu.sync_copy(zbuf_v, table_sh.at[pl.ds(s * ROWS_PER_TILE, ROWS_PER_TILE)])
    pltpu.make_async_copy(eidx_hbm.at[c, pl.ds(s * TPT, TPT)], idx_v, isem).wait()
    plsc.subcore_barrier()

    def start_scatter(b, j):
        pltpu.async_copy(ones_v, table_sh.at[idx_v.at[j]], ssem.at[b], add=True)

    def wait_scatter(b, j):
        pltpu.make_async_copy(ones_v, table_sh.at[idx_v.at[j]], ssem.at[b]).wait()

    for b in range(DEGW):
        start_scatter(b, b)

    @pl.loop(DEGW, TPT, step=DEGW)
    def _(j0):
        for b in range(DEGW):
            wait_scatter(b, j0 - DEGW + b)
            start_scatter(b, j0 + b)

    for b in range(DEGW):
        wait_scatter(b, TPT - DEGW + b)

    @pl.when(s < TEX)
    def _():
        pltpu.sync_copy(eidx_hbm.at[c, pl.ds(NS * TPT + s, 1)], exi_v)
        pltpu.sync_copy(ones_v, table_sh.at[exi_v.at[0]], add=True)

    plsc.subcore_barrier()
    pltpu.sync_copy(
        table_sh.at[pl.ds(s * ROWS_PER_TILE, ROWS_PER_TILE)],
        deg_hbm.at[c, pl.ds(s * ROWS_PER_TILE, ROWS_PER_TILE)],
    )


def _mp_pipeline(gather_src, sidx, didx, gbuf, table_sh, gsem, ssem, nchunks, K):
    ngroups = nchunks // K

    def start_gather(p, b, j):
        pltpu.async_copy(gather_src.at[sidx.at[j]], gbuf.at[p, b], gsem.at[p, b])

    def wait_gather(p, b, j):
        pltpu.make_async_copy(
            gather_src.at[sidx.at[j]], gbuf.at[p, b], gsem.at[p, b]
        ).wait()

    def start_scatter(p, b, j):
        pltpu.async_copy(
            gbuf.at[p, b], table_sh.at[didx.at[j]], ssem.at[p, b], add=True
        )

    def wait_scatter(p, b, j):
        pltpu.make_async_copy(
            gbuf.at[p, b], table_sh.at[didx.at[j]], ssem.at[p, b]
        ).wait()

    for b in range(K):
        start_gather(0, b, b)

    @pl.loop(0, ngroups, step=2)
    def _(t):
        j0 = t * K
        j1 = j0 + K
        for b in range(K):
            wait_gather(0, b, j0 + b)

        @pl.when(t > 0)
        def _():
            for b in range(K):
                wait_scatter(1, b, j0 - K + b)

        for b in range(K):
            start_gather(1, b, j1 + b)
        for b in range(K):
            start_scatter(0, b, j0 + b)
        for b in range(K):
            wait_gather(1, b, j1 + b)
        for b in range(K):
            wait_scatter(0, b, j0 + b)

        @pl.when(t + 2 < ngroups)
        def _():
            for b in range(K):
                start_gather(0, b, j1 + K + b)

        for b in range(K):
            start_scatter(1, b, j1 + b)

    for b in range(K):
        wait_scatter(1, b, (ngroups - 1) * K + b)


@functools.partial(
    pl.kernel,
    out_type=jax.ShapeDtypeStruct((NC, TABN, D), _f32),
    mesh=_mesh,
    compiler_params=_sc_params,
    scratch_types=[
        pltpu.VMEM((P1, CK), jnp.int32),
        pltpu.VMEM((P1, CK), jnp.int32),
        pltpu.VMEM((1, CK), jnp.int32),
        pltpu.VMEM((1, CK), jnp.int32),
        pltpu.VMEM((2, 1, CK, D), _f32),
        pltpu.VMEM((8, D), _f32),
        pltpu.VMEM_SHARED((TABN, D), _f32),
        pltpu.SemaphoreType.DMA((2, 1)),
        pltpu.SemaphoreType.DMA((2, 1)),
        pltpu.SemaphoreType.DMA((2,)),
    ],
)
def _mp1_kernel(
    eidx_hbm, h1_hbm, agg_hbm, sidx, didx, exs, exd, gbuf, zbuf, table_sh,
    gsem, ssem, isem,
):
    c = lax.axis_index("c")
    s = lax.axis_index("s")

    def load_idx(phase):
        base = c * HR + s * HPT + phase * P1
        pltpu.async_copy(eidx_hbm.at[0, pl.ds(base, P1)], sidx, isem.at[0])
        pltpu.async_copy(eidx_hbm.at[1, pl.ds(base, P1)], didx, isem.at[1])
        pltpu.make_async_copy(eidx_hbm.at[0, pl.ds(base, P1)], sidx, isem.at[0]).wait()
        pltpu.make_async_copy(eidx_hbm.at[1, pl.ds(base, P1)], didx, isem.at[1]).wait()

    load_idx(0)
    _fill(zbuf, 8, D, 0.0)

    @pl.loop(0, ROWS_PER_TILE // 8)
    def _(i):
        pltpu.sync_copy(zbuf, table_sh.at[pl.ds(s * ROWS_PER_TILE + i * 8, 8)])

    plsc.subcore_barrier()

    for phase in range(HPT // P1):
        if phase:
            load_idx(phase)
        _mp_pipeline(h1_hbm, sidx, didx, gbuf, table_sh, gsem, ssem, P1, 1)

    @pl.when(s < HEX)
    def _():
        r = c * HR + NS * HPT + s
        pltpu.sync_copy(eidx_hbm.at[0, pl.ds(r, 1)], exs)
        pltpu.sync_copy(eidx_hbm.at[1, pl.ds(r, 1)], exd)
        pltpu.sync_copy(h1_hbm.at[exs.at[0]], gbuf.at[0, 0])
        pltpu.sync_copy(gbuf.at[0, 0], table_sh.at[exd.at[0]], add=True)

    plsc.subcore_barrier()
    pltpu.sync_copy(
        table_sh.at[pl.ds(s * ROWS_PER_TILE, ROWS_PER_TILE)],
        agg_hbm.at[c, pl.ds(s * ROWS_PER_TILE, ROWS_PER_TILE)],
    )


@functools.partial(
    pl.kernel,
    out_type=jax.ShapeDtypeStruct((NC, TABN, 16), _f32),
    mesh=_mesh,
    compiler_params=_sc_params,
    scratch_types=[
        pltpu.VMEM((HPT, CK), jnp.int32),
        pltpu.VMEM((HPT, CK), jnp.int32),
        pltpu.VMEM((1, CK), jnp.int32),
        pltpu.VMEM((1, CK), jnp.int32),
        pltpu.VMEM((2, K2, CK, 16), _f32),
        pltpu.VMEM((8, 16), _f32),
        pltpu.VMEM_SHARED((TABN, 16), _f32),
        pltpu.SemaphoreType.DMA((2, K2)),
        pltpu.SemaphoreType.DMA((2, K2)),
        pltpu.SemaphoreType.DMA((2,)),
    ],
)
def _mp2_kernel(
    eidx_hbm, h2_hbm, agg_hbm, sidx, didx, exs, exd, gbuf, zbuf, table_sh,
    gsem, ssem, isem,
):
    c = lax.axis_index("c")
    s = lax.axis_index("s")
    base = c * HR + s * HPT
    pltpu.async_copy(eidx_hbm.at[0, pl.ds(base, HPT)], sidx, isem.at[0])
    pltpu.async_copy(eidx_hbm.at[1, pl.ds(base, HPT)], didx, isem.at[1])
    _fill(zbuf, 8, 16, 0.0)

    @pl.loop(0, ROWS_PER_TILE // 8)
    def _(i):
        pltpu.sync_copy(zbuf, table_sh.at[pl.ds(s * ROWS_PER_TILE + i * 8, 8)])

    pltpu.make_async_copy(eidx_hbm.at[0, pl.ds(base, HPT)], sidx, isem.at[0]).wait()
    pltpu.make_async_copy(eidx_hbm.at[1, pl.ds(base, HPT)], didx, isem.at[1]).wait()
    plsc.subcore_barrier()

    _mp_pipeline(h2_hbm, sidx, didx, gbuf, table_sh, gsem, ssem, HPT, K2)

    @pl.when(s < HEX)
    def _():
        r = c * HR + NS * HPT + s
        pltpu.sync_copy(eidx_hbm.at[0, pl.ds(r, 1)], exs)
        pltpu.sync_copy(eidx_hbm.at[1, pl.ds(r, 1)], exd)
        pltpu.sync_copy(h2_hbm.at[exs.at[0]], gbuf.at[0, 0])
        pltpu.sync_copy(gbuf.at[0, 0], table_sh.at[exd.at[0]], add=True)

    plsc.subcore_barrier()
    pltpu.sync_copy(
        table_sh.at[pl.ds(s * ROWS_PER_TILE, ROWS_PER_TILE)],
        agg_hbm.at[c, pl.ds(s * ROWS_PER_TILE, ROWS_PER_TILE)],
    )


_BM = 2000


def _mm1_body(deg_ref, x_ref, w_ref, out_ref):
    ns = lax.rsqrt(jnp.clip(deg_ref[0, :, 0:1], 1.0, None))
    h = x_ref[...] * ns
    out_ref[...] = jnp.dot(h, w_ref[...], preferred_element_type=_f32)


def _mm2_body(agg_ref, deg_ref, b1_ref, w2_ref, out_ref):
    ns = lax.rsqrt(jnp.clip(deg_ref[0, :, 0:1], 1.0, None))
    nd = lax.rsqrt(jnp.clip(deg_ref[1, :, 0:1], 1.0, None))
    agg = agg_ref[0] + agg_ref[1]
    t = jnp.maximum(agg * nd + b1_ref[...], 0.0) * ns
    h2 = jnp.dot(t, w2_ref[...], preferred_element_type=_f32)
    out_ref[...] = jnp.concatenate(
        [h2, jnp.zeros((h2.shape[0], 14), _f32)], axis=1
    )


def _sm_body(p_ref, deg_ref, b2_ref, out_ref):
    nd = lax.rsqrt(jnp.clip(deg_ref[1, :, 0:1], 1.0, None))
    z = (p_ref[0, :, :2] + p_ref[1, :, :2]) * nd + b2_ref[...]
    m = jnp.max(z, axis=1, keepdims=True)
    e = jnp.exp(z - m)
    out_ref[...] = e / jnp.sum(e, axis=1, keepdims=True)


def kernel(in_feat, edge_index, W1, b1, W2, b2):
    eidx3 = edge_index.reshape(2, ER, CK)

    deg = _deg_kernel(eidx3)

    grid = N // _BM
    h1 = pl.pallas_call(
        _mm1_body,
        grid=(grid,),
        in_specs=[
            pl.BlockSpec((1, _BM, 16), lambda i: (0, i, 0)),
            pl.BlockSpec((_BM, D), lambda i: (i, 0)),
            pl.BlockSpec((D, D), lambda i: (0, 0)),
        ],
        out_specs=pl.BlockSpec((_BM, D), lambda i: (i, 0)),
        out_shape=jax.ShapeDtypeStruct((TABN, D), _f32),
    )(deg, in_feat, W1)

    agg1 = _mp1_kernel(eidx3, h1)

    h2 = pl.pallas_call(
        _mm2_body,
        grid=(grid,),
        in_specs=[
            pl.BlockSpec((NC, _BM, D), lambda i: (0, i, 0)),
            pl.BlockSpec((NC, _BM, 16), lambda i: (0, i, 0)),
            pl.BlockSpec((1, D), lambda i: (0, 0)),
            pl.BlockSpec((D, 2), lambda i: (0, 0)),
        ],
        out_specs=pl.BlockSpec((_BM, 16), lambda i: (i, 0)),
        out_shape=jax.ShapeDtypeStruct((TABN, 16), _f32),
    )(agg1, deg, b1.reshape(1, D), W2)

    agg2 = _mp2_kernel(eidx3, h2)

    out = pl.pallas_call(
        _sm_body,
        grid=(grid,),
        in_specs=[
            pl.BlockSpec((NC, _BM, 16), lambda i: (0, i, 0)),
            pl.BlockSpec((NC, _BM, 16), lambda i: (0, i, 0)),
            pl.BlockSpec((1, 2), lambda i: (0, 0)),
        ],
        out_specs=pl.BlockSpec((_BM, 2), lambda i: (i, 0)),
        out_shape=jax.ShapeDtypeStruct((N, 2), _f32),
    )(agg2, deg, b2.reshape(1, 2))

    return out


# speedup vs baseline: 1.0994x; 1.0994x over previous
import functools

import jax
import jax.numpy as jnp
from jax import lax
from jax.experimental import pallas as pl
from jax.experimental.pallas import tpu as pltpu
from jax.experimental.pallas import tpu_sc as plsc

N = 10000
E = 320000
D = 128
CK = 128
ER = E // CK
TABN = 10240
NC = 2
NS = 16
ROWS_PER_TILE = TABN // NS
TPT = ER // NS
TEX = ER - NS * TPT
HR = ER // NC
HPT = HR // NS
HEX = HR - NS * HPT
P1 = 78
K1 = 3
K2 = 4
DEGW = 6

_mesh = plsc.VectorSubcoreMesh(core_axis_name="c", subcore_axis_name="s")
_sc_params = pltpu.CompilerParams(use_tc_tiling_on_sc=False)

_f32 = jnp.float32


def _fill(ref, rows, cols, value):
    vec = jnp.full((16,), value, dtype=_f32)

    @pl.loop(0, rows)
    def _(r):
        @pl.loop(0, cols, step=16)
        def _(k):
            ref[r, pl.ds(k, 16)] = vec


@functools.partial(
    pl.kernel,
    out_type=jax.ShapeDtypeStruct((NC, TABN, 16), _f32),
    mesh=_mesh,
    compiler_params=_sc_params,
    scratch_types=[
        pltpu.VMEM((TPT, CK), jnp.int32),
        pltpu.VMEM((1, CK), jnp.int32),
        pltpu.VMEM((CK, 16), _f32),
        pltpu.VMEM((ROWS_PER_TILE, 16), _f32),
        pltpu.VMEM_SHARED((TABN, 16), _f32),
        pltpu.SemaphoreType.DMA((DEGW,)),
        pltpu.SemaphoreType.DMA,
    ],
)
def _deg_kernel(eidx_hbm, deg_hbm, idx_v, exi_v, ones_v, zbuf_v, table_sh, ssem, isem):
    c = lax.axis_index("c")
    s = lax.axis_index("s")
    pltpu.async_copy(eidx_hbm.at[c, pl.ds(s * TPT, TPT)], idx_v, isem)
    _fill(ones_v, CK, 16, 1.0)
    _fill(zbuf_v, ROWS_PER_TILE, 16, 0.0)
    pltpu.sync_copy(zbuf_v, table_sh.at[pl.ds(s * ROWS_PER_TILE, ROWS_PER_TILE)])
    pltpu.make_async_copy(eidx_hbm.at[c, pl.ds(s * TPT, TPT)], idx_v, isem).wait()
    plsc.subcore_barrier()

    def start_scatter(b, j):
        pltpu.async_copy(ones_v, table_sh.at[idx_v.at[j]], ssem.at[b], add=True)

    def wait_scatter(b, j):
        pltpu.make_async_copy(ones_v, table_sh.at[idx_v.at[j]], ssem.at[b]).wait()

    for b in range(DEGW):
        start_scatter(b, b)

    @pl.loop(DEGW, TPT, step=DEGW)
    def _(j0):
        for b in range(DEGW):
            wait_scatter(b, j0 - DEGW + b)
            start_scatter(b, j0 + b)

    for b in range(DEGW):
        wait_scatter(b, TPT - DEGW + b)

    @pl.when(s < TEX)
    def _():
        pltpu.sync_copy(eidx_hbm.at[c, pl.ds(NS * TPT + s, 1)], exi_v)
        pltpu.sync_copy(ones_v, table_sh.at[exi_v.at[0]], add=True)

    plsc.subcore_barrier()
    pltpu.sync_copy(
        table_sh.at[pl.ds(s * ROWS_PER_TILE, ROWS_PER_TILE)],
        deg_hbm.at[c, pl.ds(s * ROWS_PER_TILE, ROWS_PER_TILE)],
    )


def _mp_pipeline(gather_src, sidx, didx, gbuf, table_sh, gsem, ssem, nchunks, K):
    ngroups = nchunks // K

    def start_gather(p, b, j):
        pltpu.async_copy(gather_src.at[sidx.at[j]], gbuf.at[p, b], gsem.at[p, b])

    def wait_gather(p, b, j):
        pltpu.make_async_copy(
            gather_src.at[sidx.at[j]], gbuf.at[p, b], gsem.at[p, b]
        ).wait()

    def start_scatter(p, b, j):
        pltpu.async_copy(
            gbuf.at[p, b], table_sh.at[didx.at[j]], ssem.at[p, b], add=True
        )

    def wait_scatter(p, b, j):
        pltpu.make_async_copy(
            gbuf.at[p, b], table_sh.at[didx.at[j]], ssem.at[p, b]
        ).wait()

    for b in range(K):
        start_gather(0, b, b)

    @pl.loop(0, ngroups, step=2)
    def _(t):
        j0 = t * K
        j1 = j0 + K
        for b in range(K):
            wait_gather(0, b, j0 + b)

        @pl.when(t > 0)
        def _():
            for b in range(K):
                wait_scatter(1, b, j0 - K + b)

        for b in range(K):
            start_gather(1, b, j1 + b)
        for b in range(K):
            start_scatter(0, b, j0 + b)
        for b in range(K):
            wait_gather(1, b, j1 + b)
        for b in range(K):
            wait_scatter(0, b, j0 + b)

        @pl.when(t + 2 < ngroups)
        def _():
            for b in range(K):
                start_gather(0, b, j1 + K + b)

        for b in range(K):
            start_scatter(1, b, j1 + b)

    for b in range(K):
        wait_scatter(1, b, (ngroups - 1) * K + b)


@functools.partial(
    pl.kernel,
    out_type=jax.ShapeDtypeStruct((TABN, D), _f32),
    mesh=_mesh,
    compiler_params=_sc_params,
    scratch_types=[
        pltpu.VMEM((P1, CK), jnp.int32),
        pltpu.VMEM((P1, CK), jnp.int32),
        pltpu.VMEM((1, CK), jnp.int32),
        pltpu.VMEM((1, CK), jnp.int32),
        pltpu.VMEM((2, K1, CK, D // 2), _f32),
        pltpu.VMEM((8, D // 2), _f32),
        pltpu.VMEM_SHARED((TABN, D // 2), _f32),
        pltpu.SemaphoreType.DMA((2, K1)),
        pltpu.SemaphoreType.DMA((2, K1)),
        pltpu.SemaphoreType.DMA((2,)),
    ],
)
def _mp1_kernel(
    eidx_hbm, h1v_hbm, agg_hbm, sidx, didx, exs, exd, gbuf, zbuf, table_sh,
    gsem, ssem, isem,
):
    c = lax.axis_index("c")
    s = lax.axis_index("s")

    def split_rows(ref, rows):
        @pl.loop(0, rows)
        def _(r):
            @pl.loop(0, CK, step=16)
            def _(k):
                ref[r, pl.ds(k, 16)] = ref[r, pl.ds(k, 16)] * 2 + c

    def load_idx(phase):
        base = s * TPT + phase * P1
        pltpu.async_copy(eidx_hbm.at[0, pl.ds(base, P1)], sidx, isem.at[0])
        pltpu.async_copy(eidx_hbm.at[1, pl.ds(base, P1)], didx, isem.at[1])
        pltpu.make_async_copy(eidx_hbm.at[0, pl.ds(base, P1)], sidx, isem.at[0]).wait()
        pltpu.make_async_copy(eidx_hbm.at[1, pl.ds(base, P1)], didx, isem.at[1]).wait()
        split_rows(sidx, P1)

    load_idx(0)
    _fill(zbuf, 8, D // 2, 0.0)

    @pl.loop(0, ROWS_PER_TILE // 8)
    def _(i):
        pltpu.sync_copy(zbuf, table_sh.at[pl.ds(s * ROWS_PER_TILE + i * 8, 8)])

    plsc.subcore_barrier()

    for phase in range(TPT // P1):
        if phase:
            load_idx(phase)
        _mp_pipeline(h1v_hbm, sidx, didx, gbuf, table_sh, gsem, ssem, P1, K1)

    @pl.when(s < TEX)
    def _():
        r = NS * TPT + s
        pltpu.sync_copy(eidx_hbm.at[0, pl.ds(r, 1)], exs)
        pltpu.sync_copy(eidx_hbm.at[1, pl.ds(r, 1)], exd)
        split_rows(exs, 1)
        pltpu.sync_copy(h1v_hbm.at[exs.at[0]], gbuf.at[0, 0])
        pltpu.sync_copy(gbuf.at[0, 0], table_sh.at[exd.at[0]], add=True)

    plsc.subcore_barrier()
    pltpu.sync_copy(
        table_sh.at[pl.ds(s * ROWS_PER_TILE, ROWS_PER_TILE)],
        agg_hbm.at[pl.ds(s * ROWS_PER_TILE, ROWS_PER_TILE), pl.ds(c * (D // 2), D // 2)],
    )


@functools.partial(
    pl.kernel,
    out_type=jax.ShapeDtypeStruct((NC, TABN, 16), _f32),
    mesh=_mesh,
    compiler_params=_sc_params,
    scratch_types=[
        pltpu.VMEM((HPT, CK), jnp.int32),
        pltpu.VMEM((HPT, CK), jnp.int32),
        pltpu.VMEM((1, CK), jnp.int32),
        pltpu.VMEM((1, CK), jnp.int32),
        pltpu.VMEM((2, K2, CK, 16), _f32),
        pltpu.VMEM((8, 16), _f32),
        pltpu.VMEM_SHARED((TABN, 16), _f32),
        pltpu.SemaphoreType.DMA((2, K2)),
        pltpu.SemaphoreType.DMA((2, K2)),
        pltpu.SemaphoreType.DMA((2,)),
    ],
)
def _mp2_kernel(
    eidx_hbm, h2_hbm, agg_hbm, sidx, didx, exs, exd, gbuf, zbuf, table_sh,
    gsem, ssem, isem,
):
    c = lax.axis_index("c")
    s = lax.axis_index("s")
    base = c * HR + s * HPT
    pltpu.async_copy(eidx_hbm.at[0, pl.ds(base, HPT)], sidx, isem.at[0])
    pltpu.async_copy(eidx_hbm.at[1, pl.ds(base, HPT)], didx, isem.at[1])
    _fill(zbuf, 8, 16, 0.0)

    @pl.loop(0, ROWS_PER_TILE // 8)
    def _(i):
        pltpu.sync_copy(zbuf, table_sh.at[pl.ds(s * ROWS_PER_TILE + i * 8, 8)])

    pltpu.make_async_copy(eidx_hbm.at[0, pl.ds(base, HPT)], sidx, isem.at[0]).wait()
    pltpu.make_async_copy(eidx_hbm.at[1, pl.ds(base, HPT)], didx, isem.at[1]).wait()
    plsc.subcore_barrier()

    _mp_pipeline(h2_hbm, sidx, didx, gbuf, table_sh, gsem, ssem, 72, K2)
    _mp_pipeline(h2_hbm, sidx.at[pl.ds(72, 6)], didx.at[pl.ds(72, 6)], gbuf,
                 table_sh, gsem, ssem, 6, 3)

    @pl.when(s < HEX)
    def _():
        r = c * HR + NS * HPT + s
        pltpu.sync_copy(eidx_hbm.at[0, pl.ds(r, 1)], exs)
        pltpu.sync_copy(eidx_hbm.at[1, pl.ds(r, 1)], exd)
        pltpu.sync_copy(h2_hbm.at[exs.at[0]], gbuf.at[0, 0])
        pltpu.sync_copy(gbuf.at[0, 0], table_sh.at[exd.at[0]], add=True)

    plsc.subcore_barrier()
    pltpu.sync_copy(
        table_sh.at[pl.ds(s * ROWS_PER_TILE, ROWS_PER_TILE)],
        agg_hbm.at[c, pl.ds(s * ROWS_PER_TILE, ROWS_PER_TILE)],
    )


_BM = 2000


def _mm1_body(deg_ref, x_ref, w_ref, out_ref):
    ns = lax.rsqrt(jnp.clip(deg_ref[0, :, 0:1], 1.0, None))
    h = x_ref[...] * ns
    out_ref[...] = jnp.dot(h, w_ref[...], preferred_element_type=_f32)


def _mm2_body(agg_ref, deg_ref, b1_ref, w2_ref, out_ref):
    ns = lax.rsqrt(jnp.clip(deg_ref[0, :, 0:1], 1.0, None))
    nd = lax.rsqrt(jnp.clip(deg_ref[1, :, 0:1], 1.0, None))
    agg = agg_ref[...]
    t = jnp.maximum(agg * nd + b1_ref[...], 0.0) * ns
    h2 = jnp.dot(t, w2_ref[...], preferred_element_type=_f32)
    out_ref[...] = jnp.concatenate(
        [h2, jnp.zeros((h2.shape[0], 14), _f32)], axis=1
    )


def _sm_body(p_ref, deg_ref, b2_ref, out_ref):
    nd = lax.rsqrt(jnp.clip(deg_ref[1, :, 0:1], 1.0, None))
    z = (p_ref[0, :, :2] + p_ref[1, :, :2]) * nd + b2_ref[...]
    m = jnp.max(z, axis=1, keepdims=True)
    e = jnp.exp(z - m)
    out_ref[...] = e / jnp.sum(e, axis=1, keepdims=True)


def kernel(in_feat, edge_index, W1, b1, W2, b2):
    eidx3 = edge_index.reshape(2, ER, CK)

    deg = _deg_kernel(eidx3)

    grid = N // _BM
    h1 = pl.pallas_call(
        _mm1_body,
        grid=(grid,),
        in_specs=[
            pl.BlockSpec((1, _BM, 16), lambda i: (0, i, 0)),
            pl.BlockSpec((_BM, D), lambda i: (i, 0)),
            pl.BlockSpec((D, D), lambda i: (0, 0)),
        ],
        out_specs=pl.BlockSpec((_BM, D), lambda i: (i, 0)),
        out_shape=jax.ShapeDtypeStruct((TABN, D), _f32),
    )(deg, in_feat, W1)

    agg1 = _mp1_kernel(eidx3, h1.reshape(2 * TABN, D // 2))

    h2 = pl.pallas_call(
        _mm2_body,
        grid=(grid,),
        in_specs=[
            pl.BlockSpec((_BM, D), lambda i: (i, 0)),
            pl.BlockSpec((NC, _BM, 16), lambda i: (0, i, 0)),
            pl.BlockSpec((1, D), lambda i: (0, 0)),
            pl.BlockSpec((D, 2), lambda i: (0, 0)),
        ],
        out_specs=pl.BlockSpec((_BM, 16), lambda i: (i, 0)),
        out_shape=jax.ShapeDtypeStruct((TABN, 16), _f32),
    )(agg1, deg, b1.reshape(1, D), W2)

    agg2 = _mp2_kernel(eidx3, h2)

    out = pl.pallas_call(
        _sm_body,
        grid=(grid,),
        in_specs=[
            pl.BlockSpec((NC, _BM, 16), lambda i: (0, i, 0)),
            pl.BlockSpec((NC, _BM, 16), lambda i: (0, i, 0)),
            pl.BlockSpec((1, 2), lambda i: (0, 0)),
        ],
        out_specs=pl.BlockSpec((_BM, 2), lambda i: (i, 0)),
        out_shape=jax.ShapeDtypeStruct((N, 2), _f32),
    )(agg2, deg, b2.reshape(1, 2))

    return out


# speedup vs baseline: 1.1119x; 1.0114x over previous
import functools

import jax
import jax.numpy as jnp
from jax import lax
from jax.experimental import pallas as pl
from jax.experimental.pallas import tpu as pltpu
from jax.experimental.pallas import tpu_sc as plsc

N = 10000
E = 320000
D = 128
CK = 128
ER = E // CK
TABN = 10240
NC = 2
NS = 16
ROWS_PER_TILE = TABN // NS
TPT = ER // NS
TEX = ER - NS * TPT
HR = ER // NC
HPT = HR // NS
HEX = HR - NS * HPT
P1 = 78
K1 = 3
K2 = 6
DEGW = 12

_mesh = plsc.VectorSubcoreMesh(core_axis_name="c", subcore_axis_name="s")
_sc_params = pltpu.CompilerParams(use_tc_tiling_on_sc=False)

_f32 = jnp.float32


def _fill(ref, rows, cols, value):
    vec = jnp.full((16,), value, dtype=_f32)

    @pl.loop(0, rows)
    def _(r):
        @pl.loop(0, cols, step=16)
        def _(k):
            ref[r, pl.ds(k, 16)] = vec


@functools.partial(
    pl.kernel,
    out_type=jax.ShapeDtypeStruct((NC, TABN, 16), _f32),
    mesh=_mesh,
    compiler_params=_sc_params,
    scratch_types=[
        pltpu.VMEM((TPT, CK), jnp.int32),
        pltpu.VMEM((1, CK), jnp.int32),
        pltpu.VMEM((CK, 16), _f32),
        pltpu.VMEM((ROWS_PER_TILE, 16), _f32),
        pltpu.VMEM_SHARED((TABN, 16), _f32),
        pltpu.SemaphoreType.DMA((DEGW,)),
        pltpu.SemaphoreType.DMA,
    ],
)
def _deg_kernel(eidx_hbm, deg_hbm, idx_v, exi_v, ones_v, zbuf_v, table_sh, ssem, isem):
    c = lax.axis_index("c")
    s = lax.axis_index("s")
    pltpu.async_copy(eidx_hbm.at[c, pl.ds(s * TPT, TPT)], idx_v, isem)
    _fill(ones_v, CK, 16, 1.0)
    _fill(zbuf_v, ROWS_PER_TILE, 16, 0.0)
    pltpu.sync_copy(zbuf_v, table_sh.at[pl.ds(s * ROWS_PER_TILE, ROWS_PER_TILE)])
    pltpu.make_async_copy(eidx_hbm.at[c, pl.ds(s * TPT, TPT)], idx_v, isem).wait()
    plsc.subcore_barrier()

    def start_scatter(b, j):
        pltpu.async_copy(ones_v, table_sh.at[idx_v.at[j]], ssem.at[b], add=True)

    def wait_scatter(b, j):
        pltpu.make_async_copy(ones_v, table_sh.at[idx_v.at[j]], ssem.at[b]).wait()

    for b in range(DEGW):
        start_scatter(b, b)

    @pl.loop(DEGW, TPT, step=DEGW)
    def _(j0):
        for b in range(DEGW):
            wait_scatter(b, j0 - DEGW + b)
            start_scatter(b, j0 + b)

    for b in range(DEGW):
        wait_scatter(b, TPT - DEGW + b)

    @pl.when(s < TEX)
    def _():
        pltpu.sync_copy(eidx_hbm.at[c, pl.ds(NS * TPT + s, 1)], exi_v)
        pltpu.sync_copy(ones_v, table_sh.at[exi_v.at[0]], add=True)

    plsc.subcore_barrier()
    pltpu.sync_copy(
        table_sh.at[pl.ds(s * ROWS_PER_TILE, ROWS_PER_TILE)],
        deg_hbm.at[c, pl.ds(s * ROWS_PER_TILE, ROWS_PER_TILE)],
    )


def _mp_pipeline(gather_src, sidx, didx, gbuf, table_sh, gsem, ssem, nchunks, K):
    ngroups = nchunks // K

    def start_gather(p, b, j):
        pltpu.async_copy(gather_src.at[sidx.at[j]], gbuf.at[p, b], gsem.at[p, b])

    def wait_gather(p, b, j):
        pltpu.make_async_copy(
            gather_src.at[sidx.at[j]], gbuf.at[p, b], gsem.at[p, b]
        ).wait()

    def start_scatter(p, b, j):
        pltpu.async_copy(
            gbuf.at[p, b], table_sh.at[didx.at[j]], ssem.at[p, b], add=True
        )

    def wait_scatter(p, b, j):
        pltpu.make_async_copy(
            gbuf.at[p, b], table_sh.at[didx.at[j]], ssem.at[p, b]
        ).wait()

    for b in range(K):
        start_gather(0, b, b)

    @pl.loop(0, ngroups, step=2)
    def _(t):
        j0 = t * K
        j1 = j0 + K
        for b in range(K):
            wait_gather(0, b, j0 + b)

        @pl.when(t > 0)
        def _():
            for b in range(K):
                wait_scatter(1, b, j0 - K + b)

        for b in range(K):
            start_gather(1, b, j1 + b)
        for b in range(K):
            start_scatter(0, b, j0 + b)
        for b in range(K):
            wait_gather(1, b, j1 + b)
        for b in range(K):
            wait_scatter(0, b, j0 + b)

        @pl.when(t + 2 < ngroups)
        def _():
            for b in range(K):
                start_gather(0, b, j1 + K + b)

        for b in range(K):
            start_scatter(1, b, j1 + b)

    for b in range(K):
        wait_scatter(1, b, (ngroups - 1) * K + b)


@functools.partial(
    pl.kernel,
    out_type=jax.ShapeDtypeStruct((TABN, D), _f32),
    mesh=_mesh,
    compiler_params=_sc_params,
    scratch_types=[
        pltpu.VMEM((P1, CK), jnp.int32),
        pltpu.VMEM((P1, CK), jnp.int32),
        pltpu.VMEM((1, CK), jnp.int32),
        pltpu.VMEM((1, CK), jnp.int32),
        pltpu.VMEM((2, K1, CK, D // 2), _f32),
        pltpu.VMEM((8, D // 2), _f32),
        pltpu.VMEM_SHARED((TABN, D // 2), _f32),
        pltpu.SemaphoreType.DMA((2, K1)),
        pltpu.SemaphoreType.DMA((2, K1)),
        pltpu.SemaphoreType.DMA((2,)),
    ],
)
def _mp1_kernel(
    eidx_hbm, h1v_hbm, agg_hbm, sidx, didx, exs, exd, gbuf, zbuf, table_sh,
    gsem, ssem, isem,
):
    c = lax.axis_index("c")
    s = lax.axis_index("s")

    def split_rows(ref, rows):
        @pl.loop(0, rows)
        def _(r):
            @pl.loop(0, CK, step=16)
            def _(k):
                ref[r, pl.ds(k, 16)] = ref[r, pl.ds(k, 16)] * 2 + c

    def load_idx(phase):
        base = s * TPT + phase * P1
        pltpu.async_copy(eidx_hbm.at[0, pl.ds(base, P1)], sidx, isem.at[0])
        pltpu.async_copy(eidx_hbm.at[1, pl.ds(base, P1)], didx, isem.at[1])
        pltpu.make_async_copy(eidx_hbm.at[0, pl.ds(base, P1)], sidx, isem.at[0]).wait()
        pltpu.make_async_copy(eidx_hbm.at[1, pl.ds(base, P1)], didx, isem.at[1]).wait()
        split_rows(sidx, P1)

    load_idx(0)
    _fill(zbuf, 8, D // 2, 0.0)

    @pl.loop(0, ROWS_PER_TILE // 8)
    def _(i):
        pltpu.sync_copy(zbuf, table_sh.at[pl.ds(s * ROWS_PER_TILE + i * 8, 8)])

    plsc.subcore_barrier()

    for phase in range(TPT // P1):
        if phase:
            load_idx(phase)
        _mp_pipeline(h1v_hbm, sidx, didx, gbuf, table_sh, gsem, ssem, P1, K1)

    @pl.when(s < TEX)
    def _():
        r = NS * TPT + s
        pltpu.sync_copy(eidx_hbm.at[0, pl.ds(r, 1)], exs)
        pltpu.sync_copy(eidx_hbm.at[1, pl.ds(r, 1)], exd)
        split_rows(exs, 1)
        pltpu.sync_copy(h1v_hbm.at[exs.at[0]], gbuf.at[0, 0])
        pltpu.sync_copy(gbuf.at[0, 0], table_sh.at[exd.at[0]], add=True)

    plsc.subcore_barrier()
    pltpu.sync_copy(
        table_sh.at[pl.ds(s * ROWS_PER_TILE, ROWS_PER_TILE)],
        agg_hbm.at[pl.ds(s * ROWS_PER_TILE, ROWS_PER_TILE), pl.ds(c * (D // 2), D // 2)],
    )


@functools.partial(
    pl.kernel,
    out_type=jax.ShapeDtypeStruct((NC, TABN, 16), _f32),
    mesh=_mesh,
    compiler_params=_sc_params,
    scratch_types=[
        pltpu.VMEM((HPT, CK), jnp.int32),
        pltpu.VMEM((HPT, CK), jnp.int32),
        pltpu.VMEM((1, CK), jnp.int32),
        pltpu.VMEM((1, CK), jnp.int32),
        pltpu.VMEM((2, K2, CK, 16), _f32),
        pltpu.VMEM((8, 16), _f32),
        pltpu.VMEM_SHARED((TABN, 16), _f32),
        pltpu.SemaphoreType.DMA((2, K2)),
        pltpu.SemaphoreType.DMA((2, K2)),
        pltpu.SemaphoreType.DMA((2,)),
    ],
)
def _mp2_kernel(
    eidx_hbm, h2_hbm, agg_hbm, sidx, didx, exs, exd, gbuf, zbuf, table_sh,
    gsem, ssem, isem,
):
    c = lax.axis_index("c")
    s = lax.axis_index("s")
    base = c * HR + s * HPT
    pltpu.async_copy(eidx_hbm.at[0, pl.ds(base, HPT)], sidx, isem.at[0])
    pltpu.async_copy(eidx_hbm.at[1, pl.ds(base, HPT)], didx, isem.at[1])
    _fill(zbuf, 8, 16, 0.0)

    @pl.loop(0, ROWS_PER_TILE // 8)
    def _(i):
        pltpu.sync_copy(zbuf, table_sh.at[pl.ds(s * ROWS_PER_TILE + i * 8, 8)])

    pltpu.make_async_copy(eidx_hbm.at[0, pl.ds(base, HPT)], sidx, isem.at[0]).wait()
    pltpu.make_async_copy(eidx_hbm.at[1, pl.ds(base, HPT)], didx, isem.at[1]).wait()
    plsc.subcore_barrier()

    _mp_pipeline(h2_hbm, sidx, didx, gbuf, table_sh, gsem, ssem, 72, K2)
    _mp_pipeline(h2_hbm, sidx.at[pl.ds(72, 6)], didx.at[pl.ds(72, 6)], gbuf,
                 table_sh, gsem, ssem, 6, 3)

    @pl.when(s < HEX)
    def _():
        r = c * HR + NS * HPT + s
        pltpu.sync_copy(eidx_hbm.at[0, pl.ds(r, 1)], exs)
        pltpu.sync_copy(eidx_hbm.at[1, pl.ds(r, 1)], exd)
        pltpu.sync_copy(h2_hbm.at[exs.at[0]], gbuf.at[0, 0])
        pltpu.sync_copy(gbuf.at[0, 0], table_sh.at[exd.at[0]], add=True)

    plsc.subcore_barrier()
    pltpu.sync_copy(
        table_sh.at[pl.ds(s * ROWS_PER_TILE, ROWS_PER_TILE)],
        agg_hbm.at[c, pl.ds(s * ROWS_PER_TILE, ROWS_PER_TILE)],
    )


_BM = 2000


def _mm1_body(deg_ref, x_ref, w_ref, out_ref):
    ns = lax.rsqrt(jnp.clip(deg_ref[0, :, 0:1], 1.0, None))
    h = x_ref[...] * ns
    out_ref[...] = jnp.dot(h, w_ref[...], preferred_element_type=_f32)


def _mm2_body(agg_ref, deg_ref, b1_ref, w2_ref, out_ref):
    ns = lax.rsqrt(jnp.clip(deg_ref[0, :, 0:1], 1.0, None))
    nd = lax.rsqrt(jnp.clip(deg_ref[1, :, 0:1], 1.0, None))
    agg = agg_ref[...]
    t = jnp.maximum(agg * nd + b1_ref[...], 0.0) * ns
    h2 = jnp.dot(t, w2_ref[...], preferred_element_type=_f32)
    out_ref[...] = jnp.concatenate(
        [h2, jnp.zeros((h2.shape[0], 14), _f32)], axis=1
    )


def _sm_body(p_ref, deg_ref, b2_ref, out_ref):
    nd = lax.rsqrt(jnp.clip(deg_ref[1, :, 0:1], 1.0, None))
    z = (p_ref[0, :, :2] + p_ref[1, :, :2]) * nd + b2_ref[...]
    m = jnp.max(z, axis=1, keepdims=True)
    e = jnp.exp(z - m)
    out_ref[...] = e / jnp.sum(e, axis=1, keepdims=True)


def kernel(in_feat, edge_index, W1, b1, W2, b2):
    eidx3 = edge_index.reshape(2, ER, CK)

    deg = _deg_kernel(eidx3)

    grid = N // _BM
    h1 = pl.pallas_call(
        _mm1_body,
        grid=(grid,),
        in_specs=[
            pl.BlockSpec((1, _BM, 16), lambda i: (0, i, 0)),
            pl.BlockSpec((_BM, D), lambda i: (i, 0)),
            pl.BlockSpec((D, D), lambda i: (0, 0)),
        ],
        out_specs=pl.BlockSpec((_BM, D), lambda i: (i, 0)),
        out_shape=jax.ShapeDtypeStruct((TABN, D), _f32),
    )(deg, in_feat, W1)

    agg1 = _mp1_kernel(eidx3, h1.reshape(2 * TABN, D // 2))

    h2 = pl.pallas_call(
        _mm2_body,
        grid=(grid,),
        in_specs=[
            pl.BlockSpec((_BM, D), lambda i: (i, 0)),
            pl.BlockSpec((NC, _BM, 16), lambda i: (0, i, 0)),
            pl.BlockSpec((1, D), lambda i: (0, 0)),
            pl.BlockSpec((D, 2), lambda i: (0, 0)),
        ],
        out_specs=pl.BlockSpec((_BM, 16), lambda i: (i, 0)),
        out_shape=jax.ShapeDtypeStruct((TABN, 16), _f32),
    )(agg1, deg, b1.reshape(1, D), W2)

    agg2 = _mp2_kernel(eidx3, h2)

    out = pl.pallas_call(
        _sm_body,
        grid=(grid,),
        in_specs=[
            pl.BlockSpec((NC, _BM, 16), lambda i: (0, i, 0)),
            pl.BlockSpec((NC, _BM, 16), lambda i: (0, i, 0)),
            pl.BlockSpec((1, 2), lambda i: (0, 0)),
        ],
        out_specs=pl.BlockSpec((_BM, 2), lambda i: (i, 0)),
        out_shape=jax.ShapeDtypeStruct((N, 2), _f32),
    )(agg2, deg, b2.reshape(1, 2))

    return out
